# Initial kernel scaffold; baseline (speedup 1.0000x reference)
import jax
import jax.numpy as jnp
from jax.experimental import pallas as pl


def kernel(h, pos, edge_index, t, params):
    raise NotImplementedError("write your pallas kernel here")


# speedup vs baseline: 2.1019x; 2.1019x over previous
import functools
import math

import numpy as np
import jax
import jax.numpy as jnp
from jax import lax
from jax.experimental import pallas as pl
from jax.experimental.pallas import tpu as pltpu
from jax.experimental.pallas import tpu_sc as plsc

N = 50000
E = 800000
H = 64
NPAD = 50176
EPAD = 819200
NC, NS = 2, 16
NW = NC * NS
EP_W = EPAD // NW
ROWS_T = NPAD // NS
TW = 80
CH = 8
GE = CH * 128
GH = GE // 2

_HALF = H // 2
_FREQS = np.exp(
    np.arange(_HALF, dtype=np.float32) * (-math.log(10000.0) / (_HALF - 1))
).reshape(1, _HALF)

_f32 = jnp.float32


def _silu(x):
    return x * (1.0 / (1.0 + jnp.exp(-x)))


def _b16(x):
    return x.astype(jnp.bfloat16).astype(_f32)


BN = 1568


def _init_body(hidx, tf, posp, emb, w0, b0, w1, b1, out):
    idx = hidx[...]
    embv = emb[...]
    hv = jnp.zeros((BN, H), _f32)
    for k in range(10):
        hv = hv + jnp.where(idx == k, embv[k:k + 1, :],
                            jnp.zeros((1, H), _f32))
    freqs = jnp.exp(
        lax.broadcasted_iota(jnp.int32, (1, _HALF), 1).astype(_f32)
        * (-math.log(10000.0) / (_HALF - 1)))
    ang = tf[...] * freqs
    te = jnp.concatenate([jnp.sin(ang), jnp.cos(ang)], axis=1)
    te = _silu(jnp.dot(te, w0[...], preferred_element_type=_f32) + b0[...])
    te = jnp.dot(te, w1[...], preferred_element_type=_f32) + b1[...]
    hv = hv + te
    out[:, 0:H] = hv
    out[:, H:H + 3] = posp[...][:, 0:3]
    out[:, H + 3:TW] = jnp.zeros((BN, TW - H - 3), _f32)


def _tc_init(hidx, tf, posp, emb16, w0, b0, w1, b1):
    full = lambda s: pl.BlockSpec(s, lambda i: (0,) * len(s))
    return pl.pallas_call(
        _init_body,
        grid=(NPAD // BN,),
        in_specs=[
            pl.BlockSpec((BN, 1), lambda i: (i, 0)),
            pl.BlockSpec((BN, 1), lambda i: (i, 0)),
            pl.BlockSpec((BN, 8), lambda i: (i, 0)),
            full((16, H)), full((H, H)), full((1, H)), full((H, H)), full((1, H)),
        ],
        out_specs=pl.BlockSpec((BN, TW), lambda i: (i, 0)),
        out_shape=jax.ShapeDtypeStruct((NPAD, TW), _f32),
    )(hidx, tf, posp, emb16, w0, b0, w1, b1)


BE = 2048


def _edge_body(rs, rd, w1i, w1j, w1d, b1, w2, b2, wc1, bc1, wc2t, bc2,
               m2, relw):
    hs = rs[...][:, 0:H]
    hd = rd[...][:, 0:H]
    rel = rd[...][:, H:H + 3] - rs[...][:, H:H + 3]
    d2 = jnp.sum(rel * rel, axis=1, keepdims=True)
    x = (jnp.dot(hd, w1i[...], preferred_element_type=_f32)
         + jnp.dot(hs, w1j[...], preferred_element_type=_f32)
         + _b16(d2) * _b16(w1d[...]) + b1[...])
    x = _silu(x)
    m = _silu(jnp.dot(x, w2[...], preferred_element_type=_f32) + b2[...])
    c = _silu(jnp.dot(m, wc1[...], preferred_element_type=_f32) + bc1[...])
    cw = jnp.sum(_b16(c) * _b16(wc2t[...]), axis=1, keepdims=True) + bc2[...]
    m2[0] = m[:, 0:32]
    m2[1] = m[:, 32:64]
    relw[:, 0:3] = rel * cw
    relw[:, 3:4] = jnp.ones((BE, 1), _f32)
    relw[:, 4:8] = jnp.zeros((BE, 4), _f32)


def _tc_edge(rows_s, rows_d, wts):
    full = lambda s: pl.BlockSpec(s, lambda i: (0,) * len(s))
    return pl.pallas_call(
        _edge_body,
        grid=(EPAD // BE,),
        in_specs=[
            pl.BlockSpec((BE, TW), lambda i: (i, 0)),
            pl.BlockSpec((BE, TW), lambda i: (i, 0)),
            full((H, H)), full((H, H)), full((1, H)), full((1, H)),
            full((H, H)), full((1, H)), full((H, H)), full((1, H)),
            full((1, H)), full((1, 1)),
        ],
        out_specs=[
            pl.BlockSpec((2, BE, 32), lambda i: (0, i, 0)),
            pl.BlockSpec((BE, 8), lambda i: (i, 0)),
        ],
        out_shape=[
            jax.ShapeDtypeStruct((2, EPAD, 32), _f32),
            jax.ShapeDtypeStruct((EPAD, 8), _f32),
        ],
    )(rows_s, rows_d, *wts)


def _node_body(tbl, aggm, aggp, pos0, n1h, n1a, bn1, n2, bn2, out, eps):
    h = tbl[...][:, 0:H]
    pos = tbl[...][:, H:H + 3]
    am = jnp.concatenate([aggm[0], aggm[1]], axis=1)
    x = _silu(jnp.dot(h, n1h[...], preferred_element_type=_f32)
              + jnp.dot(am, n1a[...], preferred_element_type=_f32) + bn1[...])
    h2 = h + jnp.dot(x, n2[...], preferred_element_type=_f32) + bn2[...]
    psum = aggp[0, :, 0:3] + aggp[1, :, 0:3]
    cnt = aggp[0, :, 3:4] + aggp[1, :, 3:4]
    pos2 = pos + psum / (cnt + 1.0)
    out[:, 0:H] = h2
    out[:, H:H + 3] = pos2
    out[:, H + 3:TW] = jnp.zeros((BN, TW - H - 3), _f32)
    eps[:, 0:3] = pos2 - pos0[...][:, 0:3]
    eps[:, 3:8] = jnp.zeros((BN, 5), _f32)


def _tc_node(table, aggm, aggp, pos0p, wts):
    full = lambda s: pl.BlockSpec(s, lambda i: (0,) * len(s))
    return pl.pallas_call(
        _node_body,
        grid=(NPAD // BN,),
        in_specs=[
            pl.BlockSpec((BN, TW), lambda i: (i, 0)),
            pl.BlockSpec((2, BN, 32), lambda i: (0, i, 0)),
            pl.BlockSpec((2, BN, 8), lambda i: (0, i, 0)),
            pl.BlockSpec((BN, 8), lambda i: (i, 0)),
            full((H, H)), full((H, H)), full((1, H)), full((H, H)), full((1, H)),
        ],
        out_specs=[
            pl.BlockSpec((BN, TW), lambda i: (i, 0)),
            pl.BlockSpec((BN, 8), lambda i: (i, 0)),
        ],
        out_shape=[
            jax.ShapeDtypeStruct((NPAD, TW), _f32),
            jax.ShapeDtypeStruct((NPAD, 8), _f32),
        ],
    )(table, aggm, aggp, pos0p, *wts)


@functools.lru_cache(maxsize=None)
def _mesh():
    return plsc.VectorSubcoreMesh(
        core_axis_name="c", subcore_axis_name="s",
        num_cores=NC, num_subcores=NS)


def _gather_body(table, src2, dst2, out_s, out_d,
                 idx_s, idx_d, rows_s, rows_d, sem):
    wid = lax.axis_index("c") * NS + lax.axis_index("s")
    base = wid * EP_W

    def group(g, _):
        e0 = pl.multiple_of(base + g * GE, GE)
        r0 = pl.multiple_of(e0 // 128, CH)
        pltpu.sync_copy(src2.at[pl.ds(r0, CH)], idx_s)
        pltpu.sync_copy(dst2.at[pl.ds(r0, CH)], idx_d)
        for half in range(2):
            descs = []
            for j in range(CH // 2):
                jj = half * (CH // 2) + j
                descs.append(pltpu.async_copy(
                    table.at[idx_s.at[jj]],
                    rows_s.at[pl.ds(j * 128, 128)], sem))
                descs.append(pltpu.async_copy(
                    table.at[idx_d.at[jj]],
                    rows_d.at[pl.ds(j * 128, 128)], sem))
            for d in descs:
                d.wait()
            o0 = pl.multiple_of(e0 + half * GH, GH)
            pltpu.sync_copy(rows_s, out_s.at[pl.ds(o0, GH)])
            pltpu.sync_copy(rows_d, out_d.at[pl.ds(o0, GH)])
        return ()

    lax.fori_loop(0, EP_W // GE, group, (), unroll=False)


@functools.lru_cache(maxsize=None)
def _sc_gather_fn():
    return pl.kernel(
        _gather_body,
        out_type=[
            jax.ShapeDtypeStruct((EPAD, TW), _f32),
            jax.ShapeDtypeStruct((EPAD, TW), _f32),
        ],
        mesh=_mesh(),
        compiler_params=pltpu.CompilerParams(use_tc_tiling_on_sc=False),
        scratch_types=[
            pltpu.VMEM((CH, 128), jnp.int32),
            pltpu.VMEM((CH, 128), jnp.int32),
            pltpu.VMEM((GH, TW), _f32),
            pltpu.VMEM((GH, TW), _f32),
            pltpu.SemaphoreType.DMA,
        ],
    )


def _sc_gather(table, src2, dst2):
    return _sc_gather_fn()(table, src2, dst2)


def _mscatter_body(m2, dst2, zm, aggm, idx, vals, spm_m):
    cid = lax.axis_index("c")
    sid = lax.axis_index("s")
    r0 = pl.multiple_of(sid * ROWS_T, 8)
    pltpu.sync_copy(zm.at[pl.ds(r0, ROWS_T)], spm_m.at[pl.ds(r0, ROWS_T)])
    plsc.subcore_barrier()

    def mgroup(g, _):
        e0 = pl.multiple_of(sid * (EPAD // NS) + g * GE, GE)
        pltpu.sync_copy(dst2.at[pl.ds(pl.multiple_of(e0 // 128, CH), CH)], idx)
        for half in range(2):
            h0 = pl.multiple_of(e0 + half * GH, GH)
            pltpu.sync_copy(m2.at[cid, pl.ds(h0, GH)], vals)
            for j in range(CH // 2):
                pltpu.sync_copy(vals.at[pl.ds(j * 128, 128)],
                                spm_m.at[idx.at[half * (CH // 2) + j]],
                                add=True)
        return ()

    lax.fori_loop(0, EPAD // NS // GE, mgroup, (), unroll=False)
    plsc.subcore_barrier()
    pltpu.sync_copy(spm_m.at[pl.ds(r0, ROWS_T)],
                    aggm.at[cid, pl.ds(r0, ROWS_T)])


@functools.lru_cache(maxsize=None)
def _sc_mscatter_fn():
    return pl.kernel(
        _mscatter_body,
        out_type=jax.ShapeDtypeStruct((2, NPAD, 32), _f32),
        mesh=_mesh(),
        compiler_params=pltpu.CompilerParams(use_tc_tiling_on_sc=False),
        scratch_types=[
            pltpu.VMEM((CH, 128), jnp.int32),
            pltpu.VMEM((GH, 32), _f32),
            pltpu.VMEM_SHARED((NPAD, 32), _f32),
        ],
    )


def _pscatter_body(relw, dst2, zp, aggp, idx, pvals, spm_p):
    cid = lax.axis_index("c")
    sid = lax.axis_index("s")
    r0 = pl.multiple_of(sid * ROWS_T, 8)
    pltpu.sync_copy(zp.at[pl.ds(r0, ROWS_T)], spm_p.at[pl.ds(r0, ROWS_T)])
    plsc.subcore_barrier()

    def pgroup(g, _):
        e0 = pl.multiple_of(
            cid * (EPAD // NC) + sid * (EPAD // NW) + g * GE, GE)
        pltpu.sync_copy(dst2.at[pl.ds(pl.multiple_of(e0 // 128, CH), CH)], idx)
        pltpu.sync_copy(relw.at[pl.ds(e0, GE)], pvals)
        for j in range(CH):
            pltpu.sync_copy(pvals.at[pl.ds(j * 128, 128)],
                            spm_p.at[idx.at[j]], add=True)
        return ()

    lax.fori_loop(0, EPAD // NW // GE, pgroup, (), unroll=False)
    plsc.subcore_barrier()
    pltpu.sync_copy(spm_p.at[pl.ds(r0, ROWS_T)],
                    aggp.at[cid, pl.ds(r0, ROWS_T)])


@functools.lru_cache(maxsize=None)
def _sc_pscatter_fn():
    return pl.kernel(
        _pscatter_body,
        out_type=jax.ShapeDtypeStruct((2, NPAD, 8), _f32),
        mesh=_mesh(),
        compiler_params=pltpu.CompilerParams(use_tc_tiling_on_sc=False),
        scratch_types=[
            pltpu.VMEM((CH, 128), jnp.int32),
            pltpu.VMEM((GE, 8), _f32),
            pltpu.VMEM_SHARED((NPAD, 8), _f32),
        ],
    )


def _sc_scatter(m2, relw, dst2, zm, zp):
    aggm = _sc_mscatter_fn()(m2, dst2, zm)
    aggp = _sc_pscatter_fn()(relw, dst2, zp)
    return aggm, aggp


def kernel(h, pos, edge_index, t, params):
    src = edge_index[0].astype(jnp.int32)
    dst = edge_index[1].astype(jnp.int32)
    pad_e = jnp.full((EPAD - E,), N, jnp.int32)
    src2 = jnp.concatenate([src, pad_e]).reshape(EPAD // 128, 128)
    dst2 = jnp.concatenate([dst, pad_e]).reshape(EPAD // 128, 128)

    hidx = jnp.pad(h.astype(jnp.int32), (0, NPAD - N)).reshape(NPAD, 1)
    tf = jnp.pad(t.astype(_f32), (0, NPAD - N)).reshape(NPAD, 1)
    posp = jnp.pad(pos, ((0, NPAD - N), (0, 5)))

    p = params
    emb16 = jnp.pad(p["embedding"], ((0, 16 - p["embedding"].shape[0]), (0, 0)))
    t0, t1 = p["t_mlp"]
    table = _tc_init(hidx, tf, posp, emb16,
                     t0["w"], t0["b"].reshape(1, H),
                     t1["w"], t1["b"].reshape(1, H))

    zm = jnp.zeros((NPAD, 32), _f32)
    zp = jnp.zeros((NPAD, 8), _f32)

    eps = None
    for lp in p["layers"]:
        e1w, e1b = lp["e1"]["w"], lp["e1"]["b"]
        edge_wts = (
            e1w[0:H], e1w[H:2 * H], e1w[2 * H:2 * H + 1], e1b.reshape(1, H),
            lp["e2"]["w"], lp["e2"]["b"].reshape(1, H),
            lp["c1"]["w"], lp["c1"]["b"].reshape(1, H),
            lp["c2"]["w"].reshape(1, H), lp["c2"]["b"].reshape(1, 1),
        )
        n1w = lp["n1"]["w"]
        node_wts = (n1w[0:H], n1w[H:2 * H], lp["n1"]["b"].reshape(1, H),
                    lp["n2"]["w"], lp["n2"]["b"].reshape(1, H))

        rows_s, rows_d = _sc_gather(table, src2, dst2)
        m2, relw = _tc_edge(rows_s, rows_d, edge_wts)
        aggm, aggp = _sc_scatter(m2, relw, dst2, zm, zp)
        table, eps = _tc_node(table, aggm, aggp, posp, node_wts)

    return eps[:N, 0:3]
